# Initial kernel scaffold; baseline (speedup 1.0000x reference)
import functools
import numpy as np
import jax
import jax.numpy as jnp
from jax.experimental import pallas as pl
from jax.experimental.pallas import tpu as pltpu

N = 4096
D = 512
L = 4
K = 10
TEMP_INV = 100.0
V_DECAY = 0.95
BN = 256
NB = N // BN
LANES = 128

_w = V_DECAY ** np.arange(K, dtype=np.float64)
_decay = np.zeros((1, LANES), dtype=np.float32)
_decay[0, :K] = (_w / np.abs(_w).sum()).astype(np.float32)


def _loss_kernel(idx_ref, f_ref, negs_ref, tcol_ref, trow_ref, out_ref, tops_ref):
    nb = pl.program_id(0)
    l = pl.program_id(1)

    f = f_ref[...]
    fn = f / jnp.maximum(jnp.sqrt(jnp.sum(f * f, axis=1, keepdims=True)), 1e-12)

    g = negs_ref[0]
    gn = g / jnp.maximum(jnp.sqrt(jnp.sum(g * g, axis=1, keepdims=True)), 1e-12)

    scores = jax.lax.dot_general(
        fn, gn, (((1,), (1,)), ((), ())),
        preferred_element_type=jnp.float32,
        precision=jax.lax.Precision.HIGHEST,
    )

    neg_inf = jnp.float32(-jnp.inf)
    same = tcol_ref[...] == trow_ref[...]
    is_idx = l == idx_ref[0]
    scores = jnp.where(jnp.logical_and(is_idx, same), neg_inf, scores)

    cols = jax.lax.broadcasted_iota(jnp.int32, (BN, N), 1)
    tops = []
    for _ in range(K):
        mx = jnp.max(scores, axis=1, keepdims=True)
        tops.append(mx)
        eq = scores == mx
        first = jnp.min(jnp.where(eq, cols, N), axis=1, keepdims=True)
        scores = jnp.where(cols == first, neg_inf, scores)
    top_blk = jnp.concatenate(tops + [jnp.zeros((BN, LANES - K), jnp.float32)], axis=1)
    tops_ref[pl.ds(l, 1)] = top_blk[None]

    @pl.when(l == L - 1)
    def _():
        x = tops_ref[...]
        m = jnp.max(x, axis=0)
        z = (x - m[None]) * TEMP_INV
        e = jnp.exp(z)
        s1 = jnp.sum(e, axis=0)
        s2 = jnp.sum(e * z, axis=0)
        ent = s2 / s1 - jnp.log(s1)
        decay = jnp.asarray(_decay)
        part = jnp.sum(ent * decay) * (1.0 / N)

        @pl.when(nb == 0)
        def _():
            out_ref[0, 0] = part + jnp.log(jnp.float32(L))

        @pl.when(nb > 0)
        def _():
            out_ref[0, 0] += part


@jax.jit
def _run(feature, target, negative_features, idx):
    idx_s = jnp.asarray(idx, jnp.int32).reshape(1)
    tcol = target.astype(jnp.int32).reshape(N, 1)
    trow = target.astype(jnp.int32).reshape(1, N)

    out = pl.pallas_call(
        _loss_kernel,
        grid=(NB, L),
        in_specs=[
            pl.BlockSpec(memory_space=pltpu.SMEM),
            pl.BlockSpec((BN, D), lambda nb, l: (nb, 0)),
            pl.BlockSpec((1, N, D), lambda nb, l: (l, 0, 0)),
            pl.BlockSpec((BN, 1), lambda nb, l: (nb, 0)),
            pl.BlockSpec((1, N), lambda nb, l: (0, 0)),
        ],
        out_specs=pl.BlockSpec((1, 1), lambda nb, l: (0, 0)),
        out_shape=jax.ShapeDtypeStruct((1, 1), jnp.float32),
        scratch_shapes=[pltpu.VMEM((L, BN, LANES), jnp.float32)],
    )(idx_s, feature, negative_features, tcol, trow)
    return out[0, 0]


def kernel(feature, target, negative_features, idx):
    return _run(feature, target, negative_features, idx)


# speedup vs baseline: 12.5917x; 12.5917x over previous
import functools
import numpy as np
import jax
import jax.numpy as jnp
from jax.experimental import pallas as pl
from jax.experimental.pallas import tpu as pltpu

N = 4096
D = 512
L = 4
K = 10
TEMP_INV = 100.0
V_DECAY = 0.95
BN = 256
NB = N // BN
LANES = 128

_DECAY_NORM = float(1.0 / np.sum(V_DECAY ** np.arange(K, dtype=np.float64)))
_LOG_V = float(np.log(V_DECAY))


def _loss_kernel(idx_ref, f_ref, negs_ref, tcol_ref, trow_ref, out_ref, tops_ref):
    nb = pl.program_id(0)
    l = pl.program_id(1)

    f = f_ref[...]
    fn = f / jnp.maximum(jnp.sqrt(jnp.sum(f * f, axis=1, keepdims=True)), 1e-12)

    g = negs_ref[0]
    gn = g / jnp.maximum(jnp.sqrt(jnp.sum(g * g, axis=1, keepdims=True)), 1e-12)

    scores = jax.lax.dot_general(
        fn, gn, (((1,), (1,)), ((), ())),
        preferred_element_type=jnp.float32,
        precision=jax.lax.Precision.HIGHEST,
    )

    neg_inf = jnp.float32(-jnp.inf)
    same = tcol_ref[...] == trow_ref[...]
    is_idx = l == idx_ref[0]
    scores = jnp.where(jnp.logical_and(is_idx, same), neg_inf, scores)

    cols = jax.lax.broadcasted_iota(jnp.int32, (BN, N), 1)
    tops = []
    for _ in range(K):
        mx = jnp.max(scores, axis=1, keepdims=True)
        tops.append(mx)
        eq = scores == mx
        first = jnp.min(jnp.where(eq, cols, N), axis=1, keepdims=True)
        scores = jnp.where(cols == first, neg_inf, scores)
    top_blk = jnp.concatenate(tops + [jnp.zeros((BN, LANES - K), jnp.float32)], axis=1)
    tops_ref[pl.ds(l, 1)] = top_blk[None]

    @pl.when(l == L - 1)
    def _():
        x = tops_ref[...]
        m = jnp.max(x, axis=0)
        z = (x - m[None]) * TEMP_INV
        e = jnp.exp(z)
        s1 = jnp.sum(e, axis=0)
        s2 = jnp.sum(e * z, axis=0)
        ent = s2 / s1 - jnp.log(s1)
        lane = jax.lax.broadcasted_iota(jnp.int32, (1, LANES), 1)
        decay = jnp.where(
            lane < K,
            jnp.exp(lane.astype(jnp.float32) * _LOG_V) * _DECAY_NORM,
            0.0,
        )
        part = jnp.sum(ent * decay, keepdims=True).reshape(1, 1) * (1.0 / N)

        @pl.when(nb == 0)
        def _():
            out_ref[...] = part + jnp.log(jnp.float32(L))

        @pl.when(nb > 0)
        def _():
            out_ref[...] += part


@jax.jit
def _run(feature, target, negative_features, idx):
    idx_s = jnp.asarray(idx, jnp.int32).reshape(1)
    tcol = target.astype(jnp.int32).reshape(N, 1)
    trow = target.astype(jnp.int32).reshape(1, N)

    out = pl.pallas_call(
        _loss_kernel,
        grid=(NB, L),
        in_specs=[
            pl.BlockSpec(memory_space=pltpu.SMEM),
            pl.BlockSpec((BN, D), lambda nb, l: (nb, 0)),
            pl.BlockSpec((1, N, D), lambda nb, l: (l, 0, 0)),
            pl.BlockSpec((BN, 1), lambda nb, l: (nb, 0)),
            pl.BlockSpec((1, N), lambda nb, l: (0, 0)),
        ],
        out_specs=pl.BlockSpec((1, 1), lambda nb, l: (0, 0)),
        out_shape=jax.ShapeDtypeStruct((1, 1), jnp.float32),
        scratch_shapes=[pltpu.VMEM((L, BN, LANES), jnp.float32)],
    )(idx_s, feature, negative_features, tcol, trow)
    return out[0, 0]


def kernel(feature, target, negative_features, idx):
    return _run(feature, target, negative_features, idx)


# speedup vs baseline: 18.0460x; 1.4332x over previous
import functools
import numpy as np
import jax
import jax.numpy as jnp
from jax.experimental import pallas as pl
from jax.experimental.pallas import tpu as pltpu

N = 4096
D = 512
L = 4
K = 10
TEMP_INV = 100.0
V_DECAY = 0.95
BN = 256
NB = N // BN
LANES = 128

_DECAY_NORM = float(1.0 / np.sum(V_DECAY ** np.arange(K, dtype=np.float64)))
_LOG_V = float(np.log(V_DECAY))


def _loss_kernel(idx_ref, f_ref, negs_ref, tcol_ref, trow_ref, out_ref, tops_ref):
    nb = pl.program_id(0)
    l = pl.program_id(1)

    f = f_ref[...]
    fn = f / jnp.maximum(jnp.sqrt(jnp.sum(f * f, axis=1, keepdims=True)), 1e-12)

    g = negs_ref[0]
    gn = g / jnp.maximum(jnp.sqrt(jnp.sum(g * g, axis=1, keepdims=True)), 1e-12)

    scores = jax.lax.dot_general(
        fn, gn, (((1,), (1,)), ((), ())),
        preferred_element_type=jnp.float32,
        precision=jax.lax.Precision.DEFAULT,
    )

    neg_inf = jnp.float32(-jnp.inf)
    same = tcol_ref[...] == trow_ref[...]
    is_idx = l == idx_ref[0]
    scores = jnp.where(jnp.logical_and(is_idx, same), neg_inf, scores)

    cols = jax.lax.broadcasted_iota(jnp.int32, (BN, N), 1)
    tops = []
    for _ in range(K):
        mx = jnp.max(scores, axis=1, keepdims=True)
        tops.append(mx)
        eq = scores == mx
        first = jnp.min(jnp.where(eq, cols, N), axis=1, keepdims=True)
        scores = jnp.where(cols == first, neg_inf, scores)
    top_blk = jnp.concatenate(tops + [jnp.zeros((BN, LANES - K), jnp.float32)], axis=1)
    tops_ref[pl.ds(l, 1)] = top_blk[None]

    @pl.when(l == L - 1)
    def _():
        x = tops_ref[...]
        m = jnp.max(x, axis=0)
        z = (x - m[None]) * TEMP_INV
        e = jnp.exp(z)
        s1 = jnp.sum(e, axis=0)
        s2 = jnp.sum(e * z, axis=0)
        ent = s2 / s1 - jnp.log(s1)
        lane = jax.lax.broadcasted_iota(jnp.int32, (1, LANES), 1)
        decay = jnp.where(
            lane < K,
            jnp.exp(lane.astype(jnp.float32) * _LOG_V) * _DECAY_NORM,
            0.0,
        )
        part = jnp.sum(ent * decay, keepdims=True).reshape(1, 1) * (1.0 / N)

        @pl.when(nb == 0)
        def _():
            out_ref[...] = part + jnp.log(jnp.float32(L))

        @pl.when(nb > 0)
        def _():
            out_ref[...] += part


@jax.jit
def _run(feature, target, negative_features, idx):
    idx_s = jnp.asarray(idx, jnp.int32).reshape(1)
    tcol = target.astype(jnp.int32).reshape(N, 1)
    trow = target.astype(jnp.int32).reshape(1, N)

    out = pl.pallas_call(
        _loss_kernel,
        grid=(NB, L),
        in_specs=[
            pl.BlockSpec(memory_space=pltpu.SMEM),
            pl.BlockSpec((BN, D), lambda nb, l: (nb, 0)),
            pl.BlockSpec((1, N, D), lambda nb, l: (l, 0, 0)),
            pl.BlockSpec((BN, 1), lambda nb, l: (nb, 0)),
            pl.BlockSpec((1, N), lambda nb, l: (0, 0)),
        ],
        out_specs=pl.BlockSpec((1, 1), lambda nb, l: (0, 0)),
        out_shape=jax.ShapeDtypeStruct((1, 1), jnp.float32),
        scratch_shapes=[pltpu.VMEM((L, BN, LANES), jnp.float32)],
    )(idx_s, feature, negative_features, tcol, trow)
    return out[0, 0]


def kernel(feature, target, negative_features, idx):
    return _run(feature, target, negative_features, idx)


# speedup vs baseline: 23.5167x; 1.3032x over previous
import functools
import numpy as np
import jax
import jax.numpy as jnp
from jax.experimental import pallas as pl
from jax.experimental.pallas import tpu as pltpu

N = 4096
D = 512
L = 4
K = 10
TEMP_INV = 100.0
V_DECAY = 0.95
BN = 256
NB = N // BN
LANES = 128

_DECAY_NORM = float(1.0 / np.sum(V_DECAY ** np.arange(K, dtype=np.float64)))
_LOG_V = float(np.log(V_DECAY))


def _loss_kernel(idx_ref, f_ref, negs_ref, tcol_ref, trow_ref, out_ref, tops_ref):
    nb = pl.program_id(0)
    l = pl.program_id(1)

    f = f_ref[...]
    fn = f / jnp.maximum(jnp.sqrt(jnp.sum(f * f, axis=1, keepdims=True)), 1e-12)

    g = negs_ref[0]
    gn = g / jnp.maximum(jnp.sqrt(jnp.sum(g * g, axis=1, keepdims=True)), 1e-12)

    scores = jax.lax.dot_general(
        fn, gn, (((1,), (1,)), ((), ())),
        preferred_element_type=jnp.float32,
        precision=jax.lax.Precision.DEFAULT,
    )

    neg_inf = jnp.float32(-jnp.inf)
    same = tcol_ref[...] == trow_ref[...]
    is_idx = l == idx_ref[0]
    scores = jnp.where(jnp.logical_and(is_idx, same), neg_inf, scores)

    cols = jax.lax.broadcasted_iota(jnp.int32, (BN, N), 1)
    b = jax.lax.bitcast_convert_type(scores, jnp.int32)
    key = b ^ jnp.bitwise_and(b >> 31, jnp.int32(0x7FFFFFFF))
    key = jnp.bitwise_or(jnp.bitwise_and(key, jnp.int32(-4096)), cols)
    int_min = jnp.int32(-(2 ** 31))
    tops = []
    for _ in range(K):
        mx = jnp.max(key, axis=1, keepdims=True)
        tops.append(mx)
        key = jnp.where(key == mx, int_min, key)
    tk = jnp.concatenate(tops, axis=1)
    tk = jnp.bitwise_and(tk, jnp.int32(-4096))
    tk = tk ^ jnp.bitwise_and(tk >> 31, jnp.int32(0x7FFFFFFF))
    vals = jax.lax.bitcast_convert_type(tk, jnp.float32)
    top_blk = jnp.concatenate([vals, jnp.zeros((BN, LANES - K), jnp.float32)], axis=1)
    tops_ref[pl.ds(l, 1)] = top_blk[None]

    @pl.when(l == L - 1)
    def _():
        x = tops_ref[...]
        m = jnp.max(x, axis=0)
        z = (x - m[None]) * TEMP_INV
        e = jnp.exp(z)
        s1 = jnp.sum(e, axis=0)
        s2 = jnp.sum(e * z, axis=0)
        ent = s2 / s1 - jnp.log(s1)
        lane = jax.lax.broadcasted_iota(jnp.int32, (1, LANES), 1)
        decay = jnp.where(
            lane < K,
            jnp.exp(lane.astype(jnp.float32) * _LOG_V) * _DECAY_NORM,
            0.0,
        )
        part = jnp.sum(ent * decay, keepdims=True).reshape(1, 1) * (1.0 / N)

        @pl.when(nb == 0)
        def _():
            out_ref[...] = part + jnp.log(jnp.float32(L))

        @pl.when(nb > 0)
        def _():
            out_ref[...] += part


@jax.jit
def _run(feature, target, negative_features, idx):
    idx_s = jnp.asarray(idx, jnp.int32).reshape(1)
    tcol = target.astype(jnp.int32).reshape(N, 1)
    trow = target.astype(jnp.int32).reshape(1, N)

    out = pl.pallas_call(
        _loss_kernel,
        grid=(NB, L),
        in_specs=[
            pl.BlockSpec(memory_space=pltpu.SMEM),
            pl.BlockSpec((BN, D), lambda nb, l: (nb, 0)),
            pl.BlockSpec((1, N, D), lambda nb, l: (l, 0, 0)),
            pl.BlockSpec((BN, 1), lambda nb, l: (nb, 0)),
            pl.BlockSpec((1, N), lambda nb, l: (0, 0)),
        ],
        out_specs=pl.BlockSpec((1, 1), lambda nb, l: (0, 0)),
        out_shape=jax.ShapeDtypeStruct((1, 1), jnp.float32),
        scratch_shapes=[pltpu.VMEM((L, BN, LANES), jnp.float32)],
    )(idx_s, feature, negative_features, tcol, trow)
    return out[0, 0]


def kernel(feature, target, negative_features, idx):
    return _run(feature, target, negative_features, idx)
